# baseline (speedup 1.0000x reference)
import functools
import jax
import jax.numpy as jnp
from jax import lax
from jax.experimental import pallas as pl
from jax.experimental.pallas import tpu as pltpu
from jax.experimental.pallas import tpu_sc as plsc

_N = 10000
_E = 320000
_H = 128
_C = 10
_L = 3

_NC = 2
_NS = 16
_NW = _NC * _NS
_EW = _E // _NW
_K = 80
_CH = _EW // _K
_NPAD = 10240
_RT = _NPAD // _NS


_FS = 4


def _rnd16(u):
    return u + jnp.uint32(0x7FFF) + ((u >> jnp.uint32(16)) & jnp.uint32(1))


def _filt_body(d_ref, wf1t_ref, bf1_ref, wf2_ref, bf2_ref, o_ref):
    wf1t = wf1t_ref[...]
    wf2 = wf2_ref[...]
    bf1c = bf1_ref[...]
    bf2 = bf2_ref[...]
    mu_c = lax.broadcasted_iota(jnp.int32, (_H, 1), 0).astype(jnp.float32) * (
        1.0 / (_H - 1))
    for j in range(_FS):
        dj = d_ref[0, j:j + 1, :]
        cut = 0.5 * (jnp.cos(jnp.pi * jnp.clip(dj, 0.0, 1.0)) + 1.0)
        dif = dj - mu_c
        ect = jnp.exp(-10.0 * dif * dif) * cut
        vt = jnp.dot(wf1t, ect, preferred_element_type=jnp.float32) + bf1c
        sspt = (jnp.maximum(vt, 0.0)
                + jnp.log(1.0 + jnp.exp(-jnp.abs(vt)))
                - jnp.log(2.0))
        y = lax.dot_general(sspt, wf2, (((0,), (0,)), ((), ())),
                            preferred_element_type=jnp.float32) + bf2
        ua = _rnd16(lax.bitcast_convert_type(y[:, :_H // 2],
                                             jnp.uint32)) >> jnp.uint32(16)
        ub = _rnd16(lax.bitcast_convert_type(y[:, _H // 2:],
                                             jnp.uint32)) & jnp.uint32(
            0xFFFF0000)
        o_ref[pl.ds(j * _H, _H), :] = lax.bitcast_convert_type(
            ua | ub, jnp.int32)


def _filt_call(d3, wf1t, bf1c, wf2, bf2):
    grid = _E // (_FS * _H)
    return pl.pallas_call(
        _filt_body,
        grid=(grid,),
        in_specs=[
            pl.BlockSpec((1, _FS, _H), lambda i: (i, 0, 0)),
            pl.BlockSpec((_H, _H), lambda i: (0, 0)),
            pl.BlockSpec((_H, 1), lambda i: (0, 0)),
            pl.BlockSpec((_H, _H), lambda i: (0, 0)),
            pl.BlockSpec((1, _H), lambda i: (0, 0)),
        ],
        out_specs=pl.BlockSpec((_FS * _H, _H // 2), lambda i: (i, 0)),
        out_shape=jax.ShapeDtypeStruct((_E, _H // 2), jnp.int32),
    )(d3, wf1t, bf1c, wf2, bf2)


def _mm_body(h_ref, w_ref, o_ref):
    o_ref[...] = jnp.dot(h_ref[...], w_ref[...],
                         preferred_element_type=jnp.float32)


def _mm_call(h, w):
    return pl.pallas_call(
        _mm_body,
        grid=(10,),
        in_specs=[
            pl.BlockSpec((_N // 10, _H), lambda i: (i, 0)),
            pl.BlockSpec((_H, _H), lambda i: (0, 0)),
        ],
        out_specs=pl.BlockSpec((_N // 10, _H), lambda i: (i, 0)),
        out_shape=jax.ShapeDtypeStruct((_N, _H), jnp.float32),
    )(h, w)


def _post_body(a0_ref, a1_ref, w_ref, o_ref):
    o_ref[...] = jnp.maximum(
        jnp.dot(a0_ref[...] + a1_ref[...], w_ref[...],
                preferred_element_type=jnp.float32), 0.0)


def _post_call(a0, a1, w):
    nb = 10
    rb = _N // nb
    return pl.pallas_call(
        _post_body,
        grid=(nb,),
        in_specs=[
            pl.BlockSpec((rb, _H), lambda i: (i, 0)),
            pl.BlockSpec((rb, _H), lambda i: (i, 0)),
            pl.BlockSpec((_H, _H), lambda i: (0, 0)),
        ],
        out_specs=pl.BlockSpec((rb, _H), lambda i: (i, 0)),
        out_shape=jax.ShapeDtypeStruct((_N, _H), jnp.float32),
    )(a0, a1, w)


def _postmm_body(a0_ref, a1_ref, wout_ref, win_ref, o_ref):
    t = jnp.maximum(
        jnp.dot(a0_ref[...] + a1_ref[...], wout_ref[...],
                preferred_element_type=jnp.float32), 0.0)
    o_ref[...] = jnp.dot(t, win_ref[...], preferred_element_type=jnp.float32)


def _postmm_call(a0, a1, wout, win_next):
    nb = 10
    rb = _N // nb
    return pl.pallas_call(
        _postmm_body,
        grid=(nb,),
        in_specs=[
            pl.BlockSpec((rb, _H), lambda i: (i, 0)),
            pl.BlockSpec((rb, _H), lambda i: (i, 0)),
            pl.BlockSpec((_H, _H), lambda i: (0, 0)),
            pl.BlockSpec((_H, _H), lambda i: (0, 0)),
        ],
        out_specs=pl.BlockSpec((rb, _H), lambda i: (i, 0)),
        out_shape=jax.ShapeDtypeStruct((_N, _H), jnp.float32),
    )(a0, a1, wout, win_next)


def _head_body(h_ref, w_ref, b_ref, o_ref, acc_ref):
    i = pl.program_id(0)

    @pl.when(i == 0)
    def _():
        acc_ref[...] = jnp.zeros_like(acc_ref)

    acc_ref[...] += jnp.sum(h_ref[...], axis=0, keepdims=True)

    @pl.when(i == pl.num_programs(0) - 1)
    def _():
        g = acc_ref[...] * (1.0 / _N)
        logits = jnp.dot(g, w_ref[...],
                         preferred_element_type=jnp.float32) + b_ref[...]
        mask = lax.broadcasted_iota(jnp.int32, (1, _H), 1) < _C
        ml = jnp.where(mask, logits, -1e30)
        m = jnp.max(ml, axis=1, keepdims=True)
        lse = m + jnp.log(
            jnp.sum(jnp.where(mask, jnp.exp(ml - m), 0.0),
                    axis=1, keepdims=True))
        o_ref[...] = logits - lse


def _head_call(h, w_pad, b_pad):
    nb = 10
    rb = _N // nb
    return pl.pallas_call(
        _head_body,
        grid=(nb,),
        in_specs=[
            pl.BlockSpec((rb, _H), lambda i: (i, 0)),
            pl.BlockSpec((_H, _H), lambda i: (0, 0)),
            pl.BlockSpec((1, _H), lambda i: (0, 0)),
        ],
        out_specs=pl.BlockSpec((1, _H), lambda i: (0, 0)),
        out_shape=jax.ShapeDtypeStruct((1, _H), jnp.float32),
        scratch_shapes=[pltpu.VMEM((1, _H), jnp.float32)],
    )(h, w_pad, b_pad)


_KH = 40
_HCH = _EW // _KH
_GRP2 = 50
_NG = _HCH // _GRP2
_NB = 4


def _sc_body(hi_hbm, filt_hbm, sd_hbm, out_hbm, sd_v, gath_v, filt_v, acc_sh,
             semg, semf, sems):
    c = lax.axis_index("c")
    s = lax.axis_index("s")
    wid = s * _NC + c

    def zrow(r, carry):
        for cc in range(_H // 16):
            gath_v[0, r, pl.ds(cc * 16, 16)] = jnp.zeros((16,), jnp.float32)
        return carry

    lax.fori_loop(0, _KH, zrow, 0)
    for j in range(_RT // _KH):
        pltpu.sync_copy(gath_v.at[0],
                        acc_sh.at[pl.ds(s * _RT + j * _KH, _KH)])
    plsc.subcore_barrier()

    def _gissue(hh, j2row, slot):
        pltpu.async_copy(hi_hbm.at[sd_v.at[j2row, 0]], gath_v.at[slot], semg)

    def _fissue(hh):
        pltpu.async_copy(
            filt_hbm.at[pl.ds(wid * _EW + hh * _KH, _KH)],
            filt_v.at[hh % 2], semf)

    def _swait():
        pltpu.make_async_copy(gath_v.at[0], acc_sh.at[sd_v.at[0, 1]],
                              sems).wait()

    def chunk(h, carry):
        grp = h // _GRP2
        j2 = h % _GRP2
        q = h % _NB

        @pl.when(j2 == 0)
        def _():
            pltpu.sync_copy(sd_hbm.at[wid, grp], sd_v)

            @pl.when(h >= 1)
            def _():
                _swait()
                _swait()
                _swait()

            _gissue(h, 0, q)
            _gissue(h + 1, 1, (h + 1) % _NB)
            _gissue(h + 2, 2, (h + 2) % _NB)

        @pl.when(h == 0)
        def _():
            _fissue(0)

        @pl.when(h + 1 < _HCH)
        def _():
            _fissue(h + 1)

        pltpu.make_async_copy(
            filt_hbm.at[pl.ds(wid * _EW + h * _KH, _KH)],
            filt_v.at[0], semf).wait()
        pltpu.make_async_copy(hi_hbm.at[sd_v.at[j2, 0]], gath_v.at[0],
                              semg).wait()

        msk = jnp.full((16,), -65536, jnp.int32)
        sh16 = jnp.full((16,), 16, jnp.int32)

        b2 = h % 2

        @plsc.parallel_loop(0, _KH, step=1, unroll=4)
        def _mult(r):
            for cc in range(4):
                x = filt_v[b2, r, pl.ds(16 * cc, 16)]
                lo = lax.bitcast_convert_type(lax.shift_left(x, sh16),
                                              jnp.float32)
                hi = lax.bitcast_convert_type(x & msk, jnp.float32)
                sla = pl.ds(16 * cc, 16)
                slb = pl.ds(64 + 16 * cc, 16)
                gath_v[q, r, sla] = gath_v[q, r, sla] * lo
                gath_v[q, r, slb] = gath_v[q, r, slb] * hi

        pltpu.async_copy(gath_v.at[q], acc_sh.at[sd_v.at[j2, 1]], sems,
                         add=True)

        @pl.when(j2 < _GRP2 - 3)
        def _():
            @pl.when(h >= 1)
            def _():
                _swait()

            _gissue(h + 3, j2 + 3, (h + 3) % _NB)

        return carry

    lax.fori_loop(0, _HCH, chunk, 0)

    for _ in range(_NB):
        _swait()
    plsc.subcore_barrier()

    pltpu.sync_copy(acc_sh.at[pl.ds(s * _RT, _RT)],
                    out_hbm.at[c * _NS + s])


@functools.cache
def _sc_pass_fn():
    mesh = plsc.VectorSubcoreMesh(core_axis_name="c", subcore_axis_name="s",
                                  num_cores=_NC, num_subcores=_NS)
    return pl.kernel(
        _sc_body,
        out_type=jax.ShapeDtypeStruct((_NW, _RT, _H), jnp.float32),
        mesh=mesh,
        scratch_types=[
            pltpu.VMEM((_GRP2, 2, _KH), jnp.int32),
            pltpu.VMEM((_NB, _KH, _H), jnp.float32),
            pltpu.VMEM((2, _KH, _H // 2), jnp.int32),
            pltpu.VMEM_SHARED((_NPAD, _H), jnp.float32),
        ] + [pltpu.SemaphoreType.DMA] * 3,
    )


def _sc_pass(hi, filt, sd5):
    return _sc_pass_fn()(hi, filt, sd5)


def kernel(x, edge_index, edge_dist, Wf1, bf1, Wf2, bf2, Win, Wout,
           fc_W, fc_b):
    sd5 = jnp.stack(
        [edge_index[0].reshape(_NW, _NG, _GRP2, _KH),
         edge_index[1].reshape(_NW, _NG, _GRP2, _KH)], axis=3)
    d2 = edge_dist.reshape(_E // (_FS * _H), _FS, _H)

    filt = _filt_call(d2, Wf1.T, bf1.reshape(_H, 1), Wf2,
                      bf2.reshape(1, _H))

    hi = _mm_call(x, Win[0])
    for i in range(_L):
        agg32 = _sc_pass(hi, filt, sd5)
        flat = agg32.reshape(2 * _NPAD, _H)
        a0, a1 = flat[:_N], flat[_NPAD:_NPAD + _N]
        if i < _L - 1:
            hi = _postmm_call(a0, a1, Wout[i], Win[i + 1])
        else:
            h = _post_call(a0, a1, Wout[i])

    w_pad = jnp.zeros((_H, _H), jnp.float32).at[:, :_C].set(fc_W)
    b_pad = jnp.zeros((1, _H), jnp.float32).at[0, :_C].set(fc_b)
    out = _head_call(h, w_pad, b_pad)
    return out[:, :_C]


# speedup vs baseline: 1.0092x; 1.0092x over previous
import functools
import jax
import jax.numpy as jnp
from jax import lax
from jax.experimental import pallas as pl
from jax.experimental.pallas import tpu as pltpu
from jax.experimental.pallas import tpu_sc as plsc

_N = 10000
_E = 320000
_H = 128
_C = 10
_L = 3

_NC = 2
_NS = 16
_NW = _NC * _NS
_EW = _E // _NW
_K = 80
_CH = _EW // _K
_NPAD = 10240
_RT = _NPAD // _NS


_FS = 4


def _rnd16(u):
    return u + jnp.uint32(0x7FFF) + ((u >> jnp.uint32(16)) & jnp.uint32(1))


def _filt_body(d_ref, wf1t_ref, bf1_ref, wf2_ref, bf2_ref, o_ref):
    wf1t = wf1t_ref[...]
    wf2 = wf2_ref[...]
    bf1c = bf1_ref[...]
    bf2 = bf2_ref[...]
    mu_c = lax.broadcasted_iota(jnp.int32, (_H, 1), 0).astype(jnp.float32) * (
        1.0 / (_H - 1))
    for j in range(_FS):
        dj = d_ref[0, j:j + 1, :]
        cut = 0.5 * (jnp.cos(jnp.pi * jnp.clip(dj, 0.0, 1.0)) + 1.0)
        dif = dj - mu_c
        ect = jnp.exp(-10.0 * dif * dif) * cut
        vt = jnp.dot(wf1t, ect, preferred_element_type=jnp.float32) + bf1c
        sspt = (jnp.maximum(vt, 0.0)
                + jnp.log(1.0 + jnp.exp(-jnp.abs(vt)))
                - jnp.log(2.0))
        y = lax.dot_general(sspt, wf2, (((0,), (0,)), ((), ())),
                            preferred_element_type=jnp.float32) + bf2
        ua = _rnd16(lax.bitcast_convert_type(y[:, :_H // 2],
                                             jnp.uint32)) >> jnp.uint32(16)
        ub = _rnd16(lax.bitcast_convert_type(y[:, _H // 2:],
                                             jnp.uint32)) & jnp.uint32(
            0xFFFF0000)
        o_ref[pl.ds(j * _H, _H), :] = lax.bitcast_convert_type(
            ua | ub, jnp.int32)


def _filt_call(d3, wf1t, bf1c, wf2, bf2):
    grid = _E // (_FS * _H)
    return pl.pallas_call(
        _filt_body,
        grid=(grid,),
        in_specs=[
            pl.BlockSpec((1, _FS, _H), lambda i: (i, 0, 0)),
            pl.BlockSpec((_H, _H), lambda i: (0, 0)),
            pl.BlockSpec((_H, 1), lambda i: (0, 0)),
            pl.BlockSpec((_H, _H), lambda i: (0, 0)),
            pl.BlockSpec((1, _H), lambda i: (0, 0)),
        ],
        out_specs=pl.BlockSpec((_FS * _H, _H // 2), lambda i: (i, 0)),
        out_shape=jax.ShapeDtypeStruct((_E, _H // 2), jnp.int32),
    )(d3, wf1t, bf1c, wf2, bf2)


def _mm_body(h_ref, w_ref, o_ref):
    o_ref[...] = jnp.dot(h_ref[...], w_ref[...],
                         preferred_element_type=jnp.float32)


def _mm_call(h, w):
    return pl.pallas_call(
        _mm_body,
        grid=(10,),
        in_specs=[
            pl.BlockSpec((_N // 10, _H), lambda i: (i, 0)),
            pl.BlockSpec((_H, _H), lambda i: (0, 0)),
        ],
        out_specs=pl.BlockSpec((_N // 10, _H), lambda i: (i, 0)),
        out_shape=jax.ShapeDtypeStruct((_N, _H), jnp.float32),
    )(h, w)


def _post_body(a0_ref, a1_ref, w_ref, o_ref):
    o_ref[...] = jnp.maximum(
        jnp.dot(a0_ref[...] + a1_ref[...], w_ref[...],
                preferred_element_type=jnp.float32), 0.0)


def _post_call(a0, a1, w):
    nb = 10
    rb = _N // nb
    return pl.pallas_call(
        _post_body,
        grid=(nb,),
        in_specs=[
            pl.BlockSpec((rb, _H), lambda i: (i, 0)),
            pl.BlockSpec((rb, _H), lambda i: (i, 0)),
            pl.BlockSpec((_H, _H), lambda i: (0, 0)),
        ],
        out_specs=pl.BlockSpec((rb, _H), lambda i: (i, 0)),
        out_shape=jax.ShapeDtypeStruct((_N, _H), jnp.float32),
    )(a0, a1, w)


def _postmm_body(a0_ref, a1_ref, wout_ref, win_ref, o_ref):
    t = jnp.maximum(
        jnp.dot(a0_ref[...] + a1_ref[...], wout_ref[...],
                preferred_element_type=jnp.float32), 0.0)
    o_ref[...] = jnp.dot(t, win_ref[...], preferred_element_type=jnp.float32)


def _postmm_call(a0, a1, wout, win_next):
    nb = 10
    rb = _N // nb
    return pl.pallas_call(
        _postmm_body,
        grid=(nb,),
        in_specs=[
            pl.BlockSpec((rb, _H), lambda i: (i, 0)),
            pl.BlockSpec((rb, _H), lambda i: (i, 0)),
            pl.BlockSpec((_H, _H), lambda i: (0, 0)),
            pl.BlockSpec((_H, _H), lambda i: (0, 0)),
        ],
        out_specs=pl.BlockSpec((rb, _H), lambda i: (i, 0)),
        out_shape=jax.ShapeDtypeStruct((_N, _H), jnp.float32),
    )(a0, a1, wout, win_next)


def _head_body(h_ref, w_ref, b_ref, o_ref, acc_ref):
    i = pl.program_id(0)

    @pl.when(i == 0)
    def _():
        acc_ref[...] = jnp.zeros_like(acc_ref)

    acc_ref[...] += jnp.sum(h_ref[...], axis=0, keepdims=True)

    @pl.when(i == pl.num_programs(0) - 1)
    def _():
        g = acc_ref[...] * (1.0 / _N)
        logits = jnp.dot(g, w_ref[...],
                         preferred_element_type=jnp.float32) + b_ref[...]
        mask = lax.broadcasted_iota(jnp.int32, (1, _H), 1) < _C
        ml = jnp.where(mask, logits, -1e30)
        m = jnp.max(ml, axis=1, keepdims=True)
        lse = m + jnp.log(
            jnp.sum(jnp.where(mask, jnp.exp(ml - m), 0.0),
                    axis=1, keepdims=True))
        o_ref[...] = logits - lse


def _head_call(h, w_pad, b_pad):
    nb = 10
    rb = _N // nb
    return pl.pallas_call(
        _head_body,
        grid=(nb,),
        in_specs=[
            pl.BlockSpec((rb, _H), lambda i: (i, 0)),
            pl.BlockSpec((_H, _H), lambda i: (0, 0)),
            pl.BlockSpec((1, _H), lambda i: (0, 0)),
        ],
        out_specs=pl.BlockSpec((1, _H), lambda i: (0, 0)),
        out_shape=jax.ShapeDtypeStruct((1, _H), jnp.float32),
        scratch_shapes=[pltpu.VMEM((1, _H), jnp.float32)],
    )(h, w_pad, b_pad)


_GRP = 25
_NG = _CH // _GRP


def _sc_body(hi_hbm, filt_hbm, sd_hbm, out_hbm,
             sd_v, gath_v, filt_v, acc_sh, semg0, semg1, semf0, semf1):
    c = lax.axis_index("c")
    s = lax.axis_index("s")
    wid = s * _NC + c

    def zrow(r, carry):
        for cc in range(_H // 16):
            gath_v[0, r, pl.ds(cc * 16, 16)] = jnp.zeros((16,), jnp.float32)
        return carry

    lax.fori_loop(0, _K, zrow, 0)
    for j in range(_RT // _K):
        pltpu.sync_copy(gath_v.at[0],
                        acc_sh.at[pl.ds(s * _RT + j * _K, _K)])
    plsc.subcore_barrier()

    def chunk(g, carry):
        grp = g // _GRP
        j = g % _GRP
        b = g % 2
        base = wid * _EW + g * _K

        @pl.when(j == 0)
        def _():
            pltpu.sync_copy(sd_hbm.at[wid, grp], sd_v)

        @pl.when((j == 0) & (b == 0))
        def _():
            pltpu.async_copy(hi_hbm.at[sd_v.at[0, 0]], gath_v.at[0], semg0)
            pltpu.async_copy(hi_hbm.at[sd_v.at[1, 0]], gath_v.at[1], semg1)

        @pl.when((j == 0) & (b == 1))
        def _():
            pltpu.async_copy(hi_hbm.at[sd_v.at[0, 0]], gath_v.at[1], semg1)
            pltpu.async_copy(hi_hbm.at[sd_v.at[1, 0]], gath_v.at[0], semg0)

        @pl.when(g == 0)
        def _():
            pltpu.async_copy(filt_hbm.at[pl.ds(base, _K)], filt_v.at[0],
                             semf0)

        @pl.when((g + 1 < _CH) & (b == 0))
        def _():
            pltpu.async_copy(filt_hbm.at[pl.ds(base + _K, _K)], filt_v.at[1],
                             semf1)

        @pl.when((g + 1 < _CH) & (b == 1))
        def _():
            pltpu.async_copy(filt_hbm.at[pl.ds(base + _K, _K)], filt_v.at[0],
                             semf0)

        @pl.when(b == 0)
        def _():
            pltpu.make_async_copy(filt_hbm.at[pl.ds(base, _K)], filt_v.at[0],
                                  semf0).wait()
            pltpu.make_async_copy(hi_hbm.at[sd_v.at[j, 0]], gath_v.at[0],
                                  semg0).wait()

        @pl.when(b == 1)
        def _():
            pltpu.make_async_copy(filt_hbm.at[pl.ds(base, _K)], filt_v.at[1],
                                  semf1).wait()
            pltpu.make_async_copy(hi_hbm.at[sd_v.at[j, 0]], gath_v.at[1],
                                  semg1).wait()

        msk = jnp.full((16,), -65536, jnp.int32)
        sh16 = jnp.full((16,), 16, jnp.int32)

        @plsc.parallel_loop(0, _K, step=1, unroll=4)
        def _mult(r):
            for cc in range(4):
                x = filt_v[b, r, pl.ds(16 * cc, 16)]
                lo = lax.bitcast_convert_type(lax.shift_left(x, sh16),
                                              jnp.float32)
                hi = lax.bitcast_convert_type(x & msk, jnp.float32)
                sla = pl.ds(16 * cc, 16)
                slb = pl.ds(64 + 16 * cc, 16)
                gath_v[b, r, sla] = gath_v[b, r, sla] * lo
                gath_v[b, r, slb] = gath_v[b, r, slb] * hi

        pltpu.sync_copy(gath_v.at[b], acc_sh.at[sd_v.at[j, 1]], add=True)

        @pl.when((j < _GRP - 2) & (b == 0))
        def _():
            pltpu.async_copy(hi_hbm.at[sd_v.at[j + 2, 0]], gath_v.at[0],
                             semg0)

        @pl.when((j < _GRP - 2) & (b == 1))
        def _():
            pltpu.async_copy(hi_hbm.at[sd_v.at[j + 2, 0]], gath_v.at[1],
                             semg1)

        return carry

    lax.fori_loop(0, _CH, chunk, 0)
    plsc.subcore_barrier()

    pltpu.sync_copy(acc_sh.at[pl.ds(s * _RT, _RT)],
                    out_hbm.at[c * _NS + s])


@functools.cache
def _sc_pass_fn():
    mesh = plsc.VectorSubcoreMesh(core_axis_name="c", subcore_axis_name="s",
                                  num_cores=_NC, num_subcores=_NS)
    return pl.kernel(
        _sc_body,
        out_type=jax.ShapeDtypeStruct((_NW, _RT, _H), jnp.float32),
        mesh=mesh,
        scratch_types=[
            pltpu.VMEM((_GRP, 2, _K), jnp.int32),
            pltpu.VMEM((2, _K, _H), jnp.float32),
            pltpu.VMEM((2, _K, _H // 2), jnp.int32),
            pltpu.VMEM_SHARED((_NPAD, _H), jnp.float32),
        ] + [pltpu.SemaphoreType.DMA] * 4,
    )


def _sc_pass(hi, filt, sd5):
    return _sc_pass_fn()(hi, filt, sd5)


def kernel(x, edge_index, edge_dist, Wf1, bf1, Wf2, bf2, Win, Wout,
           fc_W, fc_b):
    sd5 = jnp.stack(
        [edge_index[0].reshape(_NW, _NG, _GRP, _K),
         edge_index[1].reshape(_NW, _NG, _GRP, _K)], axis=3)
    d2 = edge_dist.reshape(_E // (_FS * _H), _FS, _H)

    filt = _filt_call(d2, Wf1.T, bf1.reshape(_H, 1), Wf2,
                      bf2.reshape(1, _H))

    hi = _mm_call(x, Win[0])
    for i in range(_L):
        agg32 = _sc_pass(hi, filt, sd5)
        flat = agg32.reshape(2 * _NPAD, _H)
        a0, a1 = flat[:_N], flat[_NPAD:_NPAD + _N]
        if i < _L - 1:
            hi = _postmm_call(a0, a1, Wout[i], Win[i + 1])
        else:
            h = _post_call(a0, a1, Wout[i])

    w_pad = jnp.zeros((_H, _H), jnp.float32).at[:, :_C].set(fc_W)
    b_pad = jnp.zeros((1, _H), jnp.float32).at[0, :_C].set(fc_b)
    out = _head_call(h, w_pad, b_pad)
    return out[:, :_C]


# speedup vs baseline: 1.0174x; 1.0081x over previous
import functools
import jax
import jax.numpy as jnp
from jax import lax
from jax.experimental import pallas as pl
from jax.experimental.pallas import tpu as pltpu
from jax.experimental.pallas import tpu_sc as plsc

_N = 10000
_E = 320000
_H = 128
_C = 10
_L = 3

_NC = 2
_NS = 16
_NW = _NC * _NS
_EW = _E // _NW
_K = 80
_CH = _EW // _K
_NPAD = 10240
_RT = _NPAD // _NS


_FS = 4


def _rnd16(u):
    return u + jnp.uint32(0x7FFF) + ((u >> jnp.uint32(16)) & jnp.uint32(1))


def _filt_body(d_ref, wf1t_ref, bf1_ref, wf2_ref, bf2_ref, o_ref):
    wf1t = wf1t_ref[...]
    wf2 = wf2_ref[...]
    bf1c = bf1_ref[...]
    bf2 = bf2_ref[...]
    mu_c = lax.broadcasted_iota(jnp.int32, (_H, 1), 0).astype(jnp.float32) * (
        1.0 / (_H - 1))
    for j in range(_FS):
        dj = d_ref[0, j:j + 1, :]
        cut = 0.5 * (jnp.cos(jnp.pi * jnp.clip(dj, 0.0, 1.0)) + 1.0)
        dif = dj - mu_c
        ect = jnp.exp(-10.0 * dif * dif) * cut
        vt = jnp.dot(wf1t, ect, preferred_element_type=jnp.float32) + bf1c
        sspt = (jnp.maximum(vt, 0.0)
                + jnp.log(1.0 + jnp.exp(-jnp.abs(vt)))
                - jnp.log(2.0))
        y = lax.dot_general(sspt, wf2, (((0,), (0,)), ((), ())),
                            preferred_element_type=jnp.float32) + bf2
        ua = _rnd16(lax.bitcast_convert_type(y[:, :_H // 2],
                                             jnp.uint32)) >> jnp.uint32(16)
        ub = _rnd16(lax.bitcast_convert_type(y[:, _H // 2:],
                                             jnp.uint32)) & jnp.uint32(
            0xFFFF0000)
        o_ref[pl.ds(j * _H, _H), :] = lax.bitcast_convert_type(
            ua | ub, jnp.int32)


def _filt_call(d3, wf1t, bf1c, wf2, bf2):
    grid = _E // (_FS * _H)
    return pl.pallas_call(
        _filt_body,
        grid=(grid,),
        in_specs=[
            pl.BlockSpec((1, _FS, _H), lambda i: (i, 0, 0)),
            pl.BlockSpec((_H, _H), lambda i: (0, 0)),
            pl.BlockSpec((_H, 1), lambda i: (0, 0)),
            pl.BlockSpec((_H, _H), lambda i: (0, 0)),
            pl.BlockSpec((1, _H), lambda i: (0, 0)),
        ],
        out_specs=pl.BlockSpec((_FS * _H, _H // 2), lambda i: (i, 0)),
        out_shape=jax.ShapeDtypeStruct((_E, _H // 2), jnp.int32),
    )(d3, wf1t, bf1c, wf2, bf2)


def _mm_body(h_ref, w_ref, o_ref):
    o_ref[...] = jnp.dot(h_ref[...], w_ref[...],
                         preferred_element_type=jnp.float32)


def _mm_call(h, w):
    return pl.pallas_call(
        _mm_body,
        grid=(10,),
        in_specs=[
            pl.BlockSpec((_N // 10, _H), lambda i: (i, 0)),
            pl.BlockSpec((_H, _H), lambda i: (0, 0)),
        ],
        out_specs=pl.BlockSpec((_N // 10, _H), lambda i: (i, 0)),
        out_shape=jax.ShapeDtypeStruct((_N, _H), jnp.float32),
    )(h, w)


def _post_body(a0_ref, a1_ref, w_ref, o_ref):
    o_ref[...] = jnp.maximum(
        jnp.dot(a0_ref[...] + a1_ref[...], w_ref[...],
                preferred_element_type=jnp.float32), 0.0)


def _post_call(a0, a1, w):
    nb = 10
    rb = _N // nb
    return pl.pallas_call(
        _post_body,
        grid=(nb,),
        in_specs=[
            pl.BlockSpec((rb, _H), lambda i: (i, 0)),
            pl.BlockSpec((rb, _H), lambda i: (i, 0)),
            pl.BlockSpec((_H, _H), lambda i: (0, 0)),
        ],
        out_specs=pl.BlockSpec((rb, _H), lambda i: (i, 0)),
        out_shape=jax.ShapeDtypeStruct((_N, _H), jnp.float32),
    )(a0, a1, w)


def _posthead_body(a0_ref, a1_ref, wout_ref, w_ref, b_ref, o_ref, acc_ref):
    i = pl.program_id(0)
    t = jnp.maximum(
        jnp.dot(a0_ref[...] + a1_ref[...], wout_ref[...],
                preferred_element_type=jnp.float32), 0.0)

    @pl.when(i == 0)
    def _():
        acc_ref[...] = jnp.zeros_like(acc_ref)

    acc_ref[...] += jnp.sum(t, axis=0, keepdims=True)

    @pl.when(i == pl.num_programs(0) - 1)
    def _():
        g = acc_ref[...] * (1.0 / _N)
        logits = jnp.dot(g, w_ref[...],
                         preferred_element_type=jnp.float32) + b_ref[...]
        mask = lax.broadcasted_iota(jnp.int32, (1, _H), 1) < _C
        ml = jnp.where(mask, logits, -1e30)
        m = jnp.max(ml, axis=1, keepdims=True)
        lse = m + jnp.log(
            jnp.sum(jnp.where(mask, jnp.exp(ml - m), 0.0),
                    axis=1, keepdims=True))
        o_ref[...] = logits - lse


def _posthead_call(a0, a1, wout, w_pad, b_pad):
    nb = 10
    rb = _N // nb
    return pl.pallas_call(
        _posthead_body,
        grid=(nb,),
        in_specs=[
            pl.BlockSpec((rb, _H), lambda i: (i, 0)),
            pl.BlockSpec((rb, _H), lambda i: (i, 0)),
            pl.BlockSpec((_H, _H), lambda i: (0, 0)),
            pl.BlockSpec((_H, _H), lambda i: (0, 0)),
            pl.BlockSpec((1, _H), lambda i: (0, 0)),
        ],
        out_specs=pl.BlockSpec((1, _H), lambda i: (0, 0)),
        out_shape=jax.ShapeDtypeStruct((1, _H), jnp.float32),
        scratch_shapes=[pltpu.VMEM((1, _H), jnp.float32)],
    )(a0, a1, wout, w_pad, b_pad)


def _postmm_body(a0_ref, a1_ref, wout_ref, win_ref, o_ref):
    t = jnp.maximum(
        jnp.dot(a0_ref[...] + a1_ref[...], wout_ref[...],
                preferred_element_type=jnp.float32), 0.0)
    o_ref[...] = jnp.dot(t, win_ref[...], preferred_element_type=jnp.float32)


def _postmm_call(a0, a1, wout, win_next):
    nb = 10
    rb = _N // nb
    return pl.pallas_call(
        _postmm_body,
        grid=(nb,),
        in_specs=[
            pl.BlockSpec((rb, _H), lambda i: (i, 0)),
            pl.BlockSpec((rb, _H), lambda i: (i, 0)),
            pl.BlockSpec((_H, _H), lambda i: (0, 0)),
            pl.BlockSpec((_H, _H), lambda i: (0, 0)),
        ],
        out_specs=pl.BlockSpec((rb, _H), lambda i: (i, 0)),
        out_shape=jax.ShapeDtypeStruct((_N, _H), jnp.float32),
    )(a0, a1, wout, win_next)


def _head_body(h_ref, w_ref, b_ref, o_ref, acc_ref):
    i = pl.program_id(0)

    @pl.when(i == 0)
    def _():
        acc_ref[...] = jnp.zeros_like(acc_ref)

    acc_ref[...] += jnp.sum(h_ref[...], axis=0, keepdims=True)

    @pl.when(i == pl.num_programs(0) - 1)
    def _():
        g = acc_ref[...] * (1.0 / _N)
        logits = jnp.dot(g, w_ref[...],
                         preferred_element_type=jnp.float32) + b_ref[...]
        mask = lax.broadcasted_iota(jnp.int32, (1, _H), 1) < _C
        ml = jnp.where(mask, logits, -1e30)
        m = jnp.max(ml, axis=1, keepdims=True)
        lse = m + jnp.log(
            jnp.sum(jnp.where(mask, jnp.exp(ml - m), 0.0),
                    axis=1, keepdims=True))
        o_ref[...] = logits - lse


def _head_call(h, w_pad, b_pad):
    nb = 10
    rb = _N // nb
    return pl.pallas_call(
        _head_body,
        grid=(nb,),
        in_specs=[
            pl.BlockSpec((rb, _H), lambda i: (i, 0)),
            pl.BlockSpec((_H, _H), lambda i: (0, 0)),
            pl.BlockSpec((1, _H), lambda i: (0, 0)),
        ],
        out_specs=pl.BlockSpec((1, _H), lambda i: (0, 0)),
        out_shape=jax.ShapeDtypeStruct((1, _H), jnp.float32),
        scratch_shapes=[pltpu.VMEM((1, _H), jnp.float32)],
    )(h, w_pad, b_pad)


_GRP = 25
_NG = _CH // _GRP


def _sc_body(hi_hbm, filt_hbm, sd_hbm, out_hbm,
             sd_v, gath_v, filt_v, acc_sh, semg0, semg1, semf0, semf1):
    c = lax.axis_index("c")
    s = lax.axis_index("s")
    wid = s * _NC + c

    def zrow(r, carry):
        for cc in range(_H // 16):
            gath_v[0, r, pl.ds(cc * 16, 16)] = jnp.zeros((16,), jnp.float32)
        return carry

    lax.fori_loop(0, _K, zrow, 0)
    for j in range(_RT // _K):
        pltpu.sync_copy(gath_v.at[0],
                        acc_sh.at[pl.ds(s * _RT + j * _K, _K)])
    plsc.subcore_barrier()

    def chunk(g, carry):
        grp = g // _GRP
        j = g % _GRP
        b = g % 2
        base = wid * _EW + g * _K

        @pl.when(j == 0)
        def _():
            pltpu.sync_copy(sd_hbm.at[wid, grp], sd_v)

        @pl.when((j == 0) & (b == 0))
        def _():
            pltpu.async_copy(hi_hbm.at[sd_v.at[0, 0]], gath_v.at[0], semg0)
            pltpu.async_copy(hi_hbm.at[sd_v.at[1, 0]], gath_v.at[1], semg1)

        @pl.when((j == 0) & (b == 1))
        def _():
            pltpu.async_copy(hi_hbm.at[sd_v.at[0, 0]], gath_v.at[1], semg1)
            pltpu.async_copy(hi_hbm.at[sd_v.at[1, 0]], gath_v.at[0], semg0)

        @pl.when(g == 0)
        def _():
            pltpu.async_copy(filt_hbm.at[pl.ds(base, _K)], filt_v.at[0],
                             semf0)

        @pl.when((g + 1 < _CH) & (b == 0))
        def _():
            pltpu.async_copy(filt_hbm.at[pl.ds(base + _K, _K)], filt_v.at[1],
                             semf1)

        @pl.when((g + 1 < _CH) & (b == 1))
        def _():
            pltpu.async_copy(filt_hbm.at[pl.ds(base + _K, _K)], filt_v.at[0],
                             semf0)

        @pl.when(b == 0)
        def _():
            pltpu.make_async_copy(filt_hbm.at[pl.ds(base, _K)], filt_v.at[0],
                                  semf0).wait()
            pltpu.make_async_copy(hi_hbm.at[sd_v.at[j, 0]], gath_v.at[0],
                                  semg0).wait()

        @pl.when(b == 1)
        def _():
            pltpu.make_async_copy(filt_hbm.at[pl.ds(base, _K)], filt_v.at[1],
                                  semf1).wait()
            pltpu.make_async_copy(hi_hbm.at[sd_v.at[j, 0]], gath_v.at[1],
                                  semg1).wait()

        msk = jnp.full((16,), -65536, jnp.int32)
        sh16 = jnp.full((16,), 16, jnp.int32)

        @plsc.parallel_loop(0, _K, step=1, unroll=4)
        def _mult(r):
            for cc in range(4):
                x = filt_v[b, r, pl.ds(16 * cc, 16)]
                lo = lax.bitcast_convert_type(lax.shift_left(x, sh16),
                                              jnp.float32)
                hi = lax.bitcast_convert_type(x & msk, jnp.float32)
                sla = pl.ds(16 * cc, 16)
                slb = pl.ds(64 + 16 * cc, 16)
                gath_v[b, r, sla] = gath_v[b, r, sla] * lo
                gath_v[b, r, slb] = gath_v[b, r, slb] * hi

        pltpu.sync_copy(gath_v.at[b], acc_sh.at[sd_v.at[j, 1]], add=True)

        @pl.when((j < _GRP - 2) & (b == 0))
        def _():
            pltpu.async_copy(hi_hbm.at[sd_v.at[j + 2, 0]], gath_v.at[0],
                             semg0)

        @pl.when((j < _GRP - 2) & (b == 1))
        def _():
            pltpu.async_copy(hi_hbm.at[sd_v.at[j + 2, 0]], gath_v.at[1],
                             semg1)

        return carry

    lax.fori_loop(0, _CH, chunk, 0)
    plsc.subcore_barrier()

    pltpu.sync_copy(acc_sh.at[pl.ds(s * _RT, _RT)],
                    out_hbm.at[c * _NS + s])


@functools.cache
def _sc_pass_fn():
    mesh = plsc.VectorSubcoreMesh(core_axis_name="c", subcore_axis_name="s",
                                  num_cores=_NC, num_subcores=_NS)
    return pl.kernel(
        _sc_body,
        out_type=jax.ShapeDtypeStruct((_NW, _RT, _H), jnp.float32),
        mesh=mesh,
        scratch_types=[
            pltpu.VMEM((_GRP, 2, _K), jnp.int32),
            pltpu.VMEM((2, _K, _H), jnp.float32),
            pltpu.VMEM((2, _K, _H // 2), jnp.int32),
            pltpu.VMEM_SHARED((_NPAD, _H), jnp.float32),
        ] + [pltpu.SemaphoreType.DMA] * 4,
    )


def _sc_pass(hi, filt, sd5):
    return _sc_pass_fn()(hi, filt, sd5)


def kernel(x, edge_index, edge_dist, Wf1, bf1, Wf2, bf2, Win, Wout,
           fc_W, fc_b):
    sd5 = jnp.stack(
        [edge_index[0].reshape(_NW, _NG, _GRP, _K),
         edge_index[1].reshape(_NW, _NG, _GRP, _K)], axis=3)
    d2 = edge_dist.reshape(_E // (_FS * _H), _FS, _H)

    filt = _filt_call(d2, Wf1.T, bf1.reshape(_H, 1), Wf2,
                      bf2.reshape(1, _H))

    w_pad = jnp.zeros((_H, _H), jnp.float32).at[:, :_C].set(fc_W)
    b_pad = jnp.zeros((1, _H), jnp.float32).at[0, :_C].set(fc_b)

    hi = _mm_call(x, Win[0])
    for i in range(_L):
        agg32 = _sc_pass(hi, filt, sd5)
        flat = agg32.reshape(2 * _NPAD, _H)
        a0, a1 = flat[:_N], flat[_NPAD:_NPAD + _N]
        if i < _L - 1:
            hi = _postmm_call(a0, a1, Wout[i], Win[i + 1])
        else:
            out = _posthead_call(a0, a1, Wout[i], w_pad, b_pad)
    return out[:, :_C]
